# Initial kernel scaffold; baseline (speedup 1.0000x reference)
import jax
import jax.numpy as jnp
from jax.experimental import pallas as pl


def kernel(pos, c1_l1, c1_r, c1_l2, c1_d, r1, c2_l1, c2_r, c2_l2, c2_d, r2, c3_l1, c3_r, c3_l2, c3_d, r3):
    raise NotImplementedError("write your pallas kernel here")


# speedup vs baseline: 8.6238x; 8.6238x over previous
import functools

import jax
import jax.numpy as jnp
from jax import lax
from jax.experimental import pallas as pl
from jax.experimental.pallas import tpu as pltpu
from jax.experimental.pallas import tpu_sc as plsc

_N = 4096
_K = 24
_EPS = 1e-07

_B_KNN = 128
_B_CONV = 64

_NC = 2
_NS = 16
_NW = _NC * _NS


def _knn_body(pos_r_ref, posT_ref, nbr_ref, t1_ref):
    pr = pos_r_ref[...]
    pT = posT_ref[...]
    d = None
    for i in range(3):
        diff = pr[:, i:i + 1] - pT[i:i + 1, :]
        sq = diff * diff
        d = sq if d is None else d + sq
    iota = lax.broadcasted_iota(jnp.int32, d.shape, 1)
    big = jnp.float32(jnp.inf)
    cols = []
    for _ in range(_K):
        m = jnp.min(d, axis=1, keepdims=True)
        idx = jnp.min(jnp.where(d == m, iota, _N), axis=1, keepdims=True)
        cols.append(idx)
        d = jnp.where(iota == idx, big, d)
    nbr_ref[...] = jnp.concatenate(cols, axis=1)
    pad = jnp.zeros((pr.shape[0], 125), jnp.float32)
    t1_ref[...] = jnp.concatenate([pr, pad], axis=1)


def _knn(pos3, posT):
    return pl.pallas_call(
        _knn_body,
        grid=(_N // _B_KNN,),
        in_specs=[
            pl.BlockSpec((_B_KNN, 3), lambda b: (b, 0)),
            pl.BlockSpec((3, _N), lambda b: (0, 0)),
        ],
        out_specs=[
            pl.BlockSpec((_B_KNN, _K), lambda b: (b, 0)),
            pl.BlockSpec((_B_KNN, 128), lambda b: (b, 0)),
        ],
        out_shape=[
            jax.ShapeDtypeStruct((_N, _K), jnp.int32),
            jax.ShapeDtypeStruct((_N, 128), jnp.float32),
        ],
        compiler_params=pltpu.CompilerParams(
            dimension_semantics=("parallel",)),
    )(pos3, posT)


@functools.lru_cache(maxsize=None)
def _make_sc_gather(e_total, d_row, chunk):
    rows_per_w = e_total // _NW
    n_chunks = rows_per_w // chunk
    mesh = plsc.VectorSubcoreMesh(core_axis_name="c", subcore_axis_name="s")

    @functools.partial(
        pl.kernel,
        mesh=mesh,
        out_type=jax.ShapeDtypeStruct((e_total, d_row), jnp.float32),
        scratch_types=[
            pltpu.VMEM((chunk,), jnp.int32),
            pltpu.VMEM((chunk,), jnp.int32),
            pltpu.VMEM((chunk, d_row), jnp.float32),
            pltpu.VMEM((chunk, d_row), jnp.float32),
            pltpu.SemaphoreType.DMA,
            pltpu.SemaphoreType.DMA,
        ],
    )
    def gather_k(tbl_hbm, idx_hbm, out_hbm, idx_a, idx_b, rows_a, rows_b,
                 sem_a, sem_b):
        wid = lax.axis_index("s") * _NC + lax.axis_index("c")
        base = wid * rows_per_w
        bufs = ((idx_a, rows_a, sem_a), (idx_b, rows_b, sem_b))
        pltpu.sync_copy(idx_hbm.at[pl.ds(base, chunk)], idx_a)
        cp = pltpu.async_copy(tbl_hbm.at[idx_a], rows_a, sem_a)
        for t in range(n_chunks):
            _, buf, _ = bufs[t % 2]
            cp.wait()
            if t + 1 < n_chunks:
                nidx, nbuf, nsem = bufs[(t + 1) % 2]
                pltpu.sync_copy(
                    idx_hbm.at[pl.ds(base + (t + 1) * chunk, chunk)], nidx)
                cp = pltpu.async_copy(tbl_hbm.at[nidx], nbuf, nsem)
            pltpu.sync_copy(buf, out_hbm.at[pl.ds(base + t * chunk, chunk)])

    return gather_k


def _sc_gather(table, idx):
    e_total = idx.shape[0]
    d_row = table.shape[1]
    chunk = 256 if d_row <= 128 else 192
    return _make_sc_gather(e_total, d_row, chunk)(table, idx)


def _vnrelu_parts(h, d):
    dot = h[0] * d[0] + h[1] * d[1] + h[2] * d[2]
    dsq = d[0] * d[0] + d[1] * d[1] + d[2] * d[2]
    mask = (dot >= 0.0).astype(jnp.float32)
    coef = dot / (dsq + _EPS)
    return [mask * h[i] + (1.0 - mask) * (h[i] - coef * d[i])
            for i in range(3)]


def _mm(a, b):
    return jnp.dot(a, b, preferred_element_type=jnp.float32)


def _conv_body(cin, c, has_next, ge_ref, pos_ref, wl1_ref, wr_ref, wl2_ref,
               wd_ref, r_ref, *rest):
    if has_next:
        ox_ref, oy_ref, oz_ref, tn_ref = rest
    else:
        ox_ref, oy_ref, oz_ref = rest
    x = ge_ref[...]
    pos = pos_ref[...]
    b = x.shape[1]
    rows = _K * b

    wl1 = wl1_ref[...]
    h = []
    for i in range(3):
        if cin == 1:
            hs = x[:, :, i:i + 1]
            ps = hs
        else:
            hs = x[:, :, i * cin:(i + 1) * cin]
            ps = x[:, :, 3 * cin + i:3 * cin + i + 1]
        rel = ps - pos[:, i:i + 1][None]
        hin = jnp.concatenate([hs, rel], axis=2).reshape(rows, cin + 1)
        h.append(_mm(hin, wl1))

    wr = wr_ref[...]
    d = [_mm(h[i], wr) for i in range(3)]
    h = _vnrelu_parts(h, d)

    wl2 = wl2_ref[...]
    h2 = [_mm(h[i], wl2) for i in range(3)]
    wd = wd_ref[...]
    dd = [_mm(h2[i], wd) for i in range(3)]
    dotc = (h2[0] * dd[0] + h2[1] * dd[1] + h2[2] * dd[2]).reshape(_K, b, c)
    h23 = [v.reshape(_K, b, c) for v in h2]

    best = dotc[0]
    sel = [v[0] for v in h23]
    for k in range(1, _K):
        better = dotc[k] > best
        best = jnp.where(better, dotc[k], best)
        sel = [jnp.where(better, v[k], s) for v, s in zip(h23, sel)]

    r = r_ref[...]
    dr = [_mm(sel[i], r) for i in range(3)]
    out = _vnrelu_parts(sel, dr)

    ox_ref[...] = out[0]
    oy_ref[...] = out[1]
    oz_ref[...] = out[2]
    if has_next:
        pad = jnp.zeros((b, 61), jnp.float32)
        tn_ref[...] = jnp.concatenate(out + [pos, pad], axis=1)


def _conv(ge, pos3, wl1, wr, wl2, wd, r, has_next):
    cin = wl1.shape[0] - 1
    c = wr.shape[0]
    d_row = ge.shape[2]
    grid = (_N // _B_CONV,)
    in_specs = [
        pl.BlockSpec((_K, _B_CONV, d_row), lambda bb: (0, bb, 0)),
        pl.BlockSpec((_B_CONV, 3), lambda bb: (bb, 0)),
        pl.BlockSpec((cin + 1, c), lambda bb: (0, 0)),
        pl.BlockSpec((c, c), lambda bb: (0, 0)),
        pl.BlockSpec((c, c), lambda bb: (0, 0)),
        pl.BlockSpec((c, c), lambda bb: (0, 0)),
        pl.BlockSpec((c, c), lambda bb: (0, 0)),
    ]
    out_specs = [
        pl.BlockSpec((_B_CONV, c), lambda bb: (bb, 0)),
        pl.BlockSpec((_B_CONV, c), lambda bb: (bb, 0)),
        pl.BlockSpec((_B_CONV, c), lambda bb: (bb, 0)),
    ]
    out_shape = [jax.ShapeDtypeStruct((_N, c), jnp.float32)] * 3
    if has_next:
        out_specs.append(
            pl.BlockSpec((_B_CONV, 3 * c + 64), lambda bb: (bb, 0)))
        out_shape.append(
            jax.ShapeDtypeStruct((_N, 3 * c + 64), jnp.float32))
    return pl.pallas_call(
        functools.partial(_conv_body, cin, c, has_next),
        grid=grid,
        in_specs=in_specs,
        out_specs=out_specs,
        out_shape=out_shape,
        compiler_params=pltpu.CompilerParams(
            dimension_semantics=("parallel",)),
    )(ge, pos3, wl1, wr, wl2, wd, r)


def kernel(pos, c1_l1, c1_r, c1_l2, c1_d, r1, c2_l1, c2_r, c2_l2, c2_d, r2,
           c3_l1, c3_r, c3_l2, c3_d, r3):
    c1, c2 = c1_r.shape[0], c2_r.shape[0]
    pos3 = pos[:, 0, :]
    posT = pos3.T

    nbr, t1 = _knn(pos3, posT)
    idx = nbr.T.reshape(-1)

    ge1 = _sc_gather(t1, idx).reshape(_K, _N, 128)
    h1x, h1y, h1z, t2 = _conv(
        ge1, pos3, c1_l1.T, c1_r.T, c1_l2.T, c1_d.T, r1.T, True)

    ge2 = _sc_gather(t2, idx).reshape(_K, _N, 3 * c1 + 64)
    h2x, h2y, h2z, t3 = _conv(
        ge2, pos3, c2_l1.T, c2_r.T, c2_l2.T, c2_d.T, r2.T, True)

    ge3 = _sc_gather(t3, idx).reshape(_K, _N, 3 * c2 + 64)
    h3x, h3y, h3z = _conv(
        ge3, pos3, c3_l1.T, c3_r.T, c3_l2.T, c3_d.T, r3.T, False)

    h1 = jnp.stack([h1x, h1y, h1z], axis=-1)
    h2 = jnp.stack([h2x, h2y, h2z], axis=-1)
    h3 = jnp.stack([h3x, h3y, h3z], axis=-1)
    return (h1, h2, h3)


# speedup vs baseline: 9.4427x; 1.0950x over previous
import functools

import jax
import jax.numpy as jnp
from jax import lax
from jax.experimental import pallas as pl
from jax.experimental.pallas import tpu as pltpu
from jax.experimental.pallas import tpu_sc as plsc

_N = 4096
_K = 24
_EPS = 1e-07

_B_KNN = 128
_B_CONV = 128
_H = _N // 2

_NC = 2
_NS = 16
_NW = _NC * _NS


def _knn_body(pos_r_ref, posT_ref, nbr_ref, t1_ref):
    pr = pos_r_ref[...]
    pT = posT_ref[...]
    d = None
    for i in range(3):
        diff = pr[:, i:i + 1] - pT[i:i + 1, :]
        sq = diff * diff
        d = sq if d is None else d + sq
    iota = lax.broadcasted_iota(jnp.int32, d.shape, 1)
    big = jnp.float32(jnp.inf)
    cols = []
    for _ in range(_K):
        m = jnp.min(d, axis=1, keepdims=True)
        idx = jnp.min(jnp.where(d == m, iota, _N), axis=1, keepdims=True)
        cols.append(idx)
        d = jnp.where(iota == idx, big, d)
    nbr_ref[...] = jnp.concatenate(cols, axis=1)
    pad = jnp.zeros((pr.shape[0], 125), jnp.float32)
    t1_ref[...] = jnp.concatenate([pr, pad], axis=1)


def _knn(pos3, posT):
    return pl.pallas_call(
        _knn_body,
        grid=(_N // _B_KNN,),
        in_specs=[
            pl.BlockSpec((_B_KNN, 3), lambda b: (b, 0)),
            pl.BlockSpec((3, _N), lambda b: (0, 0)),
        ],
        out_specs=[
            pl.BlockSpec((_B_KNN, _K), lambda b: (b, 0)),
            pl.BlockSpec((_B_KNN, 128), lambda b: (b, 0)),
        ],
        out_shape=[
            jax.ShapeDtypeStruct((_N, _K), jnp.int32),
            jax.ShapeDtypeStruct((_N, 128), jnp.float32),
        ],
        compiler_params=pltpu.CompilerParams(
            dimension_semantics=("parallel",)),
    )(pos3, posT)


@functools.lru_cache(maxsize=None)
def _make_sc_gather(e_total, d_row, chunk):
    rows_per_w = e_total // _NW
    n_chunks = rows_per_w // chunk
    mesh = plsc.VectorSubcoreMesh(core_axis_name="c", subcore_axis_name="s")

    @functools.partial(
        pl.kernel,
        mesh=mesh,
        out_type=jax.ShapeDtypeStruct((e_total, d_row), jnp.float32),
        scratch_types=[
            pltpu.VMEM((chunk,), jnp.int32),
            pltpu.VMEM((chunk,), jnp.int32),
            pltpu.VMEM((chunk, d_row), jnp.float32),
            pltpu.VMEM((chunk, d_row), jnp.float32),
            pltpu.SemaphoreType.DMA,
            pltpu.SemaphoreType.DMA,
        ],
    )
    def gather_k(tbl_hbm, idx_hbm, out_hbm, idx_a, idx_b, rows_a, rows_b,
                 sem_a, sem_b):
        wid = lax.axis_index("s") * _NC + lax.axis_index("c")
        base = wid * rows_per_w
        bufs = ((idx_a, rows_a, sem_a), (idx_b, rows_b, sem_b))
        pltpu.sync_copy(idx_hbm.at[pl.ds(base, chunk)], idx_a)
        cp = pltpu.async_copy(tbl_hbm.at[idx_a], rows_a, sem_a)
        for t in range(n_chunks):
            _, buf, _ = bufs[t % 2]
            cp.wait()
            if t + 1 < n_chunks:
                nidx, nbuf, nsem = bufs[(t + 1) % 2]
                pltpu.sync_copy(
                    idx_hbm.at[pl.ds(base + (t + 1) * chunk, chunk)], nidx)
                cp = pltpu.async_copy(tbl_hbm.at[nidx], nbuf, nsem)
            pltpu.sync_copy(buf, out_hbm.at[pl.ds(base + t * chunk, chunk)])

    return gather_k


def _sc_gather(table, idx):
    e_total = idx.shape[0]
    d_row = table.shape[1]
    chunk = 256 if d_row <= 128 else 192
    return _make_sc_gather(e_total, d_row, chunk)(table, idx)


def _vnrelu_parts(h, d):
    dot = h[0] * d[0] + h[1] * d[1] + h[2] * d[2]
    dsq = d[0] * d[0] + d[1] * d[1] + d[2] * d[2]
    mask = (dot >= 0.0).astype(jnp.float32)
    coef = dot / (dsq + _EPS)
    return [mask * h[i] + (1.0 - mask) * (h[i] - coef * d[i])
            for i in range(3)]


def _mm(a, b):
    return jnp.dot(a, b, preferred_element_type=jnp.float32)


def _conv_body(cin, c, has_next, ge_ref, pos_ref, wl1_ref, wr_ref, wl2_ref,
               wd_ref, r_ref, *rest):
    if has_next:
        ox_ref, oy_ref, oz_ref, tn_ref = rest
    else:
        ox_ref, oy_ref, oz_ref = rest
    x = ge_ref[...]
    pos = pos_ref[...]
    b = x.shape[1]
    rows = _K * b

    wl1 = wl1_ref[...]
    h = []
    for i in range(3):
        if cin == 1:
            hs = x[:, :, i:i + 1]
            ps = hs
        else:
            hs = x[:, :, i * cin:(i + 1) * cin]
            ps = x[:, :, 3 * cin + i:3 * cin + i + 1]
        rel = ps - pos[:, i:i + 1][None]
        hin = jnp.concatenate([hs, rel], axis=2).reshape(rows, cin + 1)
        h.append(_mm(hin, wl1))

    wr = wr_ref[...]
    d = [_mm(h[i], wr) for i in range(3)]
    h = _vnrelu_parts(h, d)

    wl2 = wl2_ref[...]
    h2 = [_mm(h[i], wl2) for i in range(3)]
    wd = wd_ref[...]
    dd = [_mm(h2[i], wd) for i in range(3)]
    dotc = (h2[0] * dd[0] + h2[1] * dd[1] + h2[2] * dd[2]).reshape(_K, b, c)
    h23 = [v.reshape(_K, b, c) for v in h2]

    best = dotc[0]
    sel = [v[0] for v in h23]
    for k in range(1, _K):
        better = dotc[k] > best
        best = jnp.where(better, dotc[k], best)
        sel = [jnp.where(better, v[k], s) for v, s in zip(h23, sel)]

    r = r_ref[...]
    dr = [_mm(sel[i], r) for i in range(3)]
    out = _vnrelu_parts(sel, dr)

    ox_ref[...] = out[0]
    oy_ref[...] = out[1]
    oz_ref[...] = out[2]
    if has_next:
        pad = jnp.zeros((b, 61), jnp.float32)
        tn_ref[...] = jnp.concatenate(out + [pos, pad], axis=1)


def _conv(ge, pos3, wl1, wr, wl2, wd, r, has_next):
    cin = wl1.shape[0] - 1
    c = wr.shape[0]
    d_row = ge.shape[2]
    n_nodes = ge.shape[1]
    grid = (n_nodes // _B_CONV,)
    in_specs = [
        pl.BlockSpec((_K, _B_CONV, d_row), lambda bb: (0, bb, 0)),
        pl.BlockSpec((_B_CONV, 3), lambda bb: (bb, 0)),
        pl.BlockSpec((cin + 1, c), lambda bb: (0, 0)),
        pl.BlockSpec((c, c), lambda bb: (0, 0)),
        pl.BlockSpec((c, c), lambda bb: (0, 0)),
        pl.BlockSpec((c, c), lambda bb: (0, 0)),
        pl.BlockSpec((c, c), lambda bb: (0, 0)),
    ]
    out_specs = [
        pl.BlockSpec((_B_CONV, c), lambda bb: (bb, 0)),
        pl.BlockSpec((_B_CONV, c), lambda bb: (bb, 0)),
        pl.BlockSpec((_B_CONV, c), lambda bb: (bb, 0)),
    ]
    out_shape = [jax.ShapeDtypeStruct((n_nodes, c), jnp.float32)] * 3
    if has_next:
        out_specs.append(
            pl.BlockSpec((_B_CONV, 3 * c + 64), lambda bb: (bb, 0)))
        out_shape.append(
            jax.ShapeDtypeStruct((n_nodes, 3 * c + 64), jnp.float32))
    return pl.pallas_call(
        functools.partial(_conv_body, cin, c, has_next),
        grid=grid,
        in_specs=in_specs,
        out_specs=out_specs,
        out_shape=out_shape,
        compiler_params=pltpu.CompilerParams(
            dimension_semantics=("parallel",)),
    )(ge, pos3, wl1, wr, wl2, wd, r)


def _layer(tbl, halves, wl1, wr, wl2, wd, r, has_next):
    outs = []
    for idx_h, pos_h in halves:
        ge = _sc_gather(tbl, idx_h).reshape(_K, _H, tbl.shape[1])
        outs.append(_conv(ge, pos_h, wl1, wr, wl2, wd, r, has_next))
    parts = [jnp.concatenate([a, b], axis=0) for a, b in zip(*outs)]
    h = jnp.stack(parts[:3], axis=-1)
    return (h, parts[3]) if has_next else (h, None)


def kernel(pos, c1_l1, c1_r, c1_l2, c1_d, r1, c2_l1, c2_r, c2_l2, c2_d, r2,
           c3_l1, c3_r, c3_l2, c3_d, r3):
    pos3 = pos[:, 0, :]
    posT = pos3.T

    nbr, t1 = _knn(pos3, posT)
    halves = tuple(
        (nbr[h * _H:(h + 1) * _H].T.reshape(-1), pos3[h * _H:(h + 1) * _H])
        for h in range(2))

    h1, t2 = _layer(t1, halves, c1_l1.T, c1_r.T, c1_l2.T, c1_d.T, r1.T, True)
    h2, t3 = _layer(t2, halves, c2_l1.T, c2_r.T, c2_l2.T, c2_d.T, r2.T, True)
    h3, _ = _layer(t3, halves, c3_l1.T, c3_r.T, c3_l2.T, c3_d.T, r3.T, False)
    return (h1, h2, h3)


# speedup vs baseline: 9.5497x; 1.0113x over previous
import functools

import jax
import jax.numpy as jnp
from jax import lax
from jax.experimental import pallas as pl
from jax.experimental.pallas import tpu as pltpu
from jax.experimental.pallas import tpu_sc as plsc

_N = 4096
_K = 24
_EPS = 1e-07

_B_KNN = 128
_B_CONV = 128
_H = _N // 2

_NC = 2
_NS = 16
_NW = _NC * _NS


def _t1_body(pos_r_ref, t1_ref):
    pr = pos_r_ref[...]
    pad = jnp.zeros((pr.shape[0], 125), jnp.float32)
    t1_ref[...] = jnp.concatenate([pr, pad], axis=1)


def _mk_t1(pos3):
    return pl.pallas_call(
        _t1_body,
        grid=(_N // 512,),
        in_specs=[pl.BlockSpec((512, 3), lambda b: (b, 0))],
        out_specs=pl.BlockSpec((512, 128), lambda b: (b, 0)),
        out_shape=jax.ShapeDtypeStruct((_N, 128), jnp.float32),
        compiler_params=pltpu.CompilerParams(
            dimension_semantics=("parallel",)),
    )(pos3)


def _knn_body(pos_r_ref, posT_ref, nbr_ref):
    pr = pos_r_ref[...]
    pT = posT_ref[...]
    d = None
    for i in range(3):
        diff = pr[:, i:i + 1] - pT[i:i + 1, :]
        sq = diff * diff
        d = sq if d is None else d + sq
    iota = lax.broadcasted_iota(jnp.int32, d.shape, 1)
    big = jnp.float32(jnp.inf)
    cols = []
    for _ in range(_K):
        m = jnp.min(d, axis=1, keepdims=True)
        idx = jnp.min(jnp.where(d == m, iota, _N), axis=1, keepdims=True)
        cols.append(idx)
        d = jnp.where(iota == idx, big, d)
    nbr_ref[...] = jnp.concatenate(cols, axis=1)


def _knn(pos_rows, posT):
    n_rows = pos_rows.shape[0]
    return pl.pallas_call(
        _knn_body,
        grid=(n_rows // _B_KNN,),
        in_specs=[
            pl.BlockSpec((_B_KNN, 3), lambda b: (b, 0)),
            pl.BlockSpec((3, _N), lambda b: (0, 0)),
        ],
        out_specs=pl.BlockSpec((_B_KNN, _K), lambda b: (b, 0)),
        out_shape=jax.ShapeDtypeStruct((n_rows, _K), jnp.int32),
        compiler_params=pltpu.CompilerParams(
            dimension_semantics=("parallel",)),
    )(pos_rows, posT)


@functools.lru_cache(maxsize=None)
def _make_sc_gather(e_total, d_row, chunk):
    rows_per_w = e_total // _NW
    n_chunks = rows_per_w // chunk
    mesh = plsc.VectorSubcoreMesh(core_axis_name="c", subcore_axis_name="s")

    @functools.partial(
        pl.kernel,
        mesh=mesh,
        out_type=jax.ShapeDtypeStruct((e_total, d_row), jnp.float32),
        scratch_types=[
            pltpu.VMEM((chunk,), jnp.int32),
            pltpu.VMEM((chunk,), jnp.int32),
            pltpu.VMEM((chunk, d_row), jnp.float32),
            pltpu.VMEM((chunk, d_row), jnp.float32),
            pltpu.SemaphoreType.DMA,
            pltpu.SemaphoreType.DMA,
        ],
    )
    def gather_k(tbl_hbm, idx_hbm, out_hbm, idx_a, idx_b, rows_a, rows_b,
                 sem_a, sem_b):
        wid = lax.axis_index("s") * _NC + lax.axis_index("c")
        base = wid * rows_per_w
        bufs = ((idx_a, rows_a, sem_a), (idx_b, rows_b, sem_b))
        pltpu.sync_copy(idx_hbm.at[pl.ds(base, chunk)], idx_a)
        cp = pltpu.async_copy(tbl_hbm.at[idx_a], rows_a, sem_a)
        for t in range(n_chunks):
            _, buf, _ = bufs[t % 2]
            cp.wait()
            if t + 1 < n_chunks:
                nidx, nbuf, nsem = bufs[(t + 1) % 2]
                pltpu.sync_copy(
                    idx_hbm.at[pl.ds(base + (t + 1) * chunk, chunk)], nidx)
                cp = pltpu.async_copy(tbl_hbm.at[nidx], nbuf, nsem)
            pltpu.sync_copy(buf, out_hbm.at[pl.ds(base + t * chunk, chunk)])

    return gather_k


def _sc_gather(table, idx):
    e_total = idx.shape[0]
    d_row = table.shape[1]
    chunk = 256 if d_row <= 128 else 192
    return _make_sc_gather(e_total, d_row, chunk)(table, idx)


def _vnrelu_parts(h, d):
    dot = h[0] * d[0] + h[1] * d[1] + h[2] * d[2]
    dsq = d[0] * d[0] + d[1] * d[1] + d[2] * d[2]
    mask = (dot >= 0.0).astype(jnp.float32)
    coef = dot / (dsq + _EPS)
    return [mask * h[i] + (1.0 - mask) * (h[i] - coef * d[i])
            for i in range(3)]


def _mm(a, b):
    return jnp.dot(a, b, preferred_element_type=jnp.float32)


def _conv_body(cin, c, has_next, ge_ref, pos_ref, wl1_ref, wr_ref, wl2_ref,
               wd_ref, r_ref, *rest):
    if has_next:
        ox_ref, oy_ref, oz_ref, tn_ref = rest
    else:
        ox_ref, oy_ref, oz_ref = rest
    x = ge_ref[...]
    pos = pos_ref[...]
    b = x.shape[1]
    rows = _K * b

    wl1 = wl1_ref[...]
    h = []
    for i in range(3):
        if cin == 1:
            hs = x[:, :, i:i + 1]
            ps = hs
        else:
            hs = x[:, :, i * cin:(i + 1) * cin]
            ps = x[:, :, 3 * cin + i:3 * cin + i + 1]
        rel = ps - pos[:, i:i + 1][None]
        hin = jnp.concatenate([hs, rel], axis=2).reshape(rows, cin + 1)
        h.append(_mm(hin, wl1))

    wr = wr_ref[...]
    d = [_mm(h[i], wr) for i in range(3)]
    h = _vnrelu_parts(h, d)

    wl2 = wl2_ref[...]
    h2 = [_mm(h[i], wl2) for i in range(3)]
    wd = wd_ref[...]
    dd = [_mm(h2[i], wd) for i in range(3)]
    dotc = (h2[0] * dd[0] + h2[1] * dd[1] + h2[2] * dd[2]).reshape(_K, b, c)
    h23 = [v.reshape(_K, b, c) for v in h2]

    best = dotc[0]
    sel = [v[0] for v in h23]
    for k in range(1, _K):
        better = dotc[k] > best
        best = jnp.where(better, dotc[k], best)
        sel = [jnp.where(better, v[k], s) for v, s in zip(h23, sel)]

    r = r_ref[...]
    dr = [_mm(sel[i], r) for i in range(3)]
    out = _vnrelu_parts(sel, dr)

    ox_ref[...] = out[0]
    oy_ref[...] = out[1]
    oz_ref[...] = out[2]
    if has_next:
        pad = jnp.zeros((b, 61), jnp.float32)
        tn_ref[...] = jnp.concatenate(out + [pos, pad], axis=1)


def _conv(ge, pos3, wl1, wr, wl2, wd, r, has_next):
    cin = wl1.shape[0] - 1
    c = wr.shape[0]
    d_row = ge.shape[2]
    n_nodes = ge.shape[1]
    grid = (n_nodes // _B_CONV,)
    in_specs = [
        pl.BlockSpec((_K, _B_CONV, d_row), lambda bb: (0, bb, 0)),
        pl.BlockSpec((_B_CONV, 3), lambda bb: (bb, 0)),
        pl.BlockSpec((cin + 1, c), lambda bb: (0, 0)),
        pl.BlockSpec((c, c), lambda bb: (0, 0)),
        pl.BlockSpec((c, c), lambda bb: (0, 0)),
        pl.BlockSpec((c, c), lambda bb: (0, 0)),
        pl.BlockSpec((c, c), lambda bb: (0, 0)),
    ]
    out_specs = [
        pl.BlockSpec((_B_CONV, c), lambda bb: (bb, 0)),
        pl.BlockSpec((_B_CONV, c), lambda bb: (bb, 0)),
        pl.BlockSpec((_B_CONV, c), lambda bb: (bb, 0)),
    ]
    out_shape = [jax.ShapeDtypeStruct((n_nodes, c), jnp.float32)] * 3
    if has_next:
        out_specs.append(
            pl.BlockSpec((_B_CONV, 3 * c + 64), lambda bb: (bb, 0)))
        out_shape.append(
            jax.ShapeDtypeStruct((n_nodes, 3 * c + 64), jnp.float32))
    return pl.pallas_call(
        functools.partial(_conv_body, cin, c, has_next),
        grid=grid,
        in_specs=in_specs,
        out_specs=out_specs,
        out_shape=out_shape,
        compiler_params=pltpu.CompilerParams(
            dimension_semantics=("parallel",)),
    )(ge, pos3, wl1, wr, wl2, wd, r)


def _layer(tbl, halves, wl1, wr, wl2, wd, r, has_next):
    outs = []
    for idx_h, pos_h in halves:
        ge = _sc_gather(tbl, idx_h).reshape(_K, _H, tbl.shape[1])
        outs.append(_conv(ge, pos_h, wl1, wr, wl2, wd, r, has_next))
    parts = [jnp.concatenate([a, b], axis=0) for a, b in zip(*outs)]
    h = jnp.stack(parts[:3], axis=-1)
    return (h, parts[3]) if has_next else (h, None)


def kernel(pos, c1_l1, c1_r, c1_l2, c1_d, r1, c2_l1, c2_r, c2_l2, c2_d, r2,
           c3_l1, c3_r, c3_l2, c3_d, r3):
    pos3 = pos[:, 0, :]
    posT = pos3.T

    t1 = _mk_t1(pos3)
    halves = tuple(
        (_knn(pos3[h * _H:(h + 1) * _H], posT).T.reshape(-1),
         pos3[h * _H:(h + 1) * _H])
        for h in range(2))

    h1, t2 = _layer(t1, halves, c1_l1.T, c1_r.T, c1_l2.T, c1_d.T, r1.T, True)
    h2, t3 = _layer(t2, halves, c2_l1.T, c2_r.T, c2_l2.T, c2_d.T, r2.T, True)
    h3, _ = _layer(t3, halves, c3_l1.T, c3_r.T, c3_l2.T, c3_d.T, r3.T, False)
    return (h1, h2, h3)


# speedup vs baseline: 9.7880x; 1.0250x over previous
import functools

import jax
import jax.numpy as jnp
from jax import lax
from jax.experimental import pallas as pl
from jax.experimental.pallas import tpu as pltpu
from jax.experimental.pallas import tpu_sc as plsc

_N = 4096
_K = 24
_EPS = 1e-07

_B_KNN = 256
_B_CONV = 128
_H = _N // 2

_NC = 2
_NS = 16
_NW = _NC * _NS


def _t1_body(pos_r_ref, t1_ref):
    pr = pos_r_ref[...]
    pad = jnp.zeros((pr.shape[0], 125), jnp.float32)
    t1_ref[...] = jnp.concatenate([pr, pad], axis=1)


def _mk_t1(pos3):
    return pl.pallas_call(
        _t1_body,
        grid=(_N // 512,),
        in_specs=[pl.BlockSpec((512, 3), lambda b: (b, 0))],
        out_specs=pl.BlockSpec((512, 128), lambda b: (b, 0)),
        out_shape=jax.ShapeDtypeStruct((_N, 128), jnp.float32),
        compiler_params=pltpu.CompilerParams(
            dimension_semantics=("parallel",)),
    )(pos3)


def _knn_body(pos_r_ref, posT_ref, nbr_ref):
    pr = pos_r_ref[...]
    pT = posT_ref[...]
    d = None
    for i in range(3):
        diff = pr[:, i:i + 1] - pT[i:i + 1, :]
        sq = diff * diff
        d = sq if d is None else d + sq
    iota = lax.broadcasted_iota(jnp.int32, d.shape, 1)
    big = jnp.float32(jnp.inf)
    cols = []
    for _ in range(_K):
        m = jnp.min(d, axis=1, keepdims=True)
        idx = jnp.min(jnp.where(d == m, iota, _N), axis=1, keepdims=True)
        cols.append(idx)
        d = jnp.where(iota == idx, big, d)
    nbr_ref[...] = jnp.concatenate(cols, axis=1)


def _knn(pos_rows, posT):
    n_rows = pos_rows.shape[0]
    return pl.pallas_call(
        _knn_body,
        grid=(n_rows // _B_KNN,),
        in_specs=[
            pl.BlockSpec((_B_KNN, 3), lambda b: (b, 0)),
            pl.BlockSpec((3, _N), lambda b: (0, 0)),
        ],
        out_specs=pl.BlockSpec((_B_KNN, _K), lambda b: (b, 0)),
        out_shape=jax.ShapeDtypeStruct((n_rows, _K), jnp.int32),
        compiler_params=pltpu.CompilerParams(
            dimension_semantics=("parallel",)),
    )(pos_rows, posT)


@functools.lru_cache(maxsize=None)
def _make_sc_gather(e_total, d_row, chunk):
    rows_per_w = e_total // _NW
    n_chunks = rows_per_w // chunk
    mesh = plsc.VectorSubcoreMesh(core_axis_name="c", subcore_axis_name="s")

    @functools.partial(
        pl.kernel,
        mesh=mesh,
        out_type=jax.ShapeDtypeStruct((e_total, d_row), jnp.float32),
        scratch_types=[
            pltpu.VMEM((chunk,), jnp.int32),
            pltpu.VMEM((chunk,), jnp.int32),
            pltpu.VMEM((chunk, d_row), jnp.float32),
            pltpu.VMEM((chunk, d_row), jnp.float32),
            pltpu.SemaphoreType.DMA,
            pltpu.SemaphoreType.DMA,
        ],
    )
    def gather_k(tbl_hbm, idx_hbm, out_hbm, idx_a, idx_b, rows_a, rows_b,
                 sem_a, sem_b):
        wid = lax.axis_index("s") * _NC + lax.axis_index("c")
        base = wid * rows_per_w
        bufs = ((idx_a, rows_a, sem_a), (idx_b, rows_b, sem_b))
        pltpu.sync_copy(idx_hbm.at[pl.ds(base, chunk)], idx_a)
        cp = pltpu.async_copy(tbl_hbm.at[idx_a], rows_a, sem_a)
        for t in range(n_chunks):
            _, buf, _ = bufs[t % 2]
            cp.wait()
            if t + 1 < n_chunks:
                nidx, nbuf, nsem = bufs[(t + 1) % 2]
                pltpu.sync_copy(
                    idx_hbm.at[pl.ds(base + (t + 1) * chunk, chunk)], nidx)
                cp = pltpu.async_copy(tbl_hbm.at[nidx], nbuf, nsem)
            pltpu.sync_copy(buf, out_hbm.at[pl.ds(base + t * chunk, chunk)])

    return gather_k


def _sc_gather(table, idx):
    e_total = idx.shape[0]
    d_row = table.shape[1]
    chunk = 256 if d_row <= 128 else 192
    return _make_sc_gather(e_total, d_row, chunk)(table, idx)


def _vnrelu_parts(h, d):
    dot = h[0] * d[0] + h[1] * d[1] + h[2] * d[2]
    dsq = d[0] * d[0] + d[1] * d[1] + d[2] * d[2]
    mask = (dot >= 0.0).astype(jnp.float32)
    coef = dot / (dsq + _EPS)
    return [mask * h[i] + (1.0 - mask) * (h[i] - coef * d[i])
            for i in range(3)]


def _mm(a, b):
    return jnp.dot(a, b, preferred_element_type=jnp.float32)


def _conv_body(cin, c, has_next, ge_ref, pos_ref, wl1_ref, wr_ref, wl2_ref,
               wd_ref, r_ref, *rest):
    if has_next:
        ox_ref, oy_ref, oz_ref, tn_ref = rest
    else:
        ox_ref, oy_ref, oz_ref = rest
    x = ge_ref[...]
    pos = pos_ref[...]
    b = x.shape[1]
    rows = _K * b

    wl1 = wl1_ref[...]
    h = []
    for i in range(3):
        if cin == 1:
            hs = x[:, :, i:i + 1]
            ps = hs
        else:
            hs = x[:, :, i * cin:(i + 1) * cin]
            ps = x[:, :, 3 * cin + i:3 * cin + i + 1]
        rel = ps - pos[:, i:i + 1][None]
        hin = jnp.concatenate([hs, rel], axis=2).reshape(rows, cin + 1)
        h.append(_mm(hin, wl1))

    wr = wr_ref[...]
    d = [_mm(h[i], wr) for i in range(3)]
    h = _vnrelu_parts(h, d)

    wl2 = wl2_ref[...]
    h2 = [_mm(h[i], wl2) for i in range(3)]
    wd = wd_ref[...]
    dd = [_mm(h2[i], wd) for i in range(3)]
    dotc = (h2[0] * dd[0] + h2[1] * dd[1] + h2[2] * dd[2]).reshape(_K, b, c)
    h23 = [v.reshape(_K, b, c) for v in h2]

    items = [(dotc[k], h23[0][k], h23[1][k], h23[2][k]) for k in range(_K)]
    while len(items) > 1:
        nxt = []
        for i in range(0, len(items) - 1, 2):
            a, bp = items[i], items[i + 1]
            better = bp[0] > a[0]
            nxt.append(tuple(jnp.where(better, y, x)
                             for x, y in zip(a, bp)))
        if len(items) % 2:
            nxt.append(items[-1])
        items = nxt
    best, sel = items[0][0], list(items[0][1:])

    r = r_ref[...]
    dr = [_mm(sel[i], r) for i in range(3)]
    out = _vnrelu_parts(sel, dr)

    ox_ref[...] = out[0]
    oy_ref[...] = out[1]
    oz_ref[...] = out[2]
    if has_next:
        pad = jnp.zeros((b, 61), jnp.float32)
        tn_ref[...] = jnp.concatenate(out + [pos, pad], axis=1)


def _conv(ge, pos3, wl1, wr, wl2, wd, r, has_next):
    cin = wl1.shape[0] - 1
    c = wr.shape[0]
    d_row = ge.shape[2]
    n_nodes = ge.shape[1]
    grid = (n_nodes // _B_CONV,)
    in_specs = [
        pl.BlockSpec((_K, _B_CONV, d_row), lambda bb: (0, bb, 0)),
        pl.BlockSpec((_B_CONV, 3), lambda bb: (bb, 0)),
        pl.BlockSpec((cin + 1, c), lambda bb: (0, 0)),
        pl.BlockSpec((c, c), lambda bb: (0, 0)),
        pl.BlockSpec((c, c), lambda bb: (0, 0)),
        pl.BlockSpec((c, c), lambda bb: (0, 0)),
        pl.BlockSpec((c, c), lambda bb: (0, 0)),
    ]
    out_specs = [
        pl.BlockSpec((_B_CONV, c), lambda bb: (bb, 0)),
        pl.BlockSpec((_B_CONV, c), lambda bb: (bb, 0)),
        pl.BlockSpec((_B_CONV, c), lambda bb: (bb, 0)),
    ]
    out_shape = [jax.ShapeDtypeStruct((n_nodes, c), jnp.float32)] * 3
    if has_next:
        out_specs.append(
            pl.BlockSpec((_B_CONV, 3 * c + 64), lambda bb: (bb, 0)))
        out_shape.append(
            jax.ShapeDtypeStruct((n_nodes, 3 * c + 64), jnp.float32))
    return pl.pallas_call(
        functools.partial(_conv_body, cin, c, has_next),
        grid=grid,
        in_specs=in_specs,
        out_specs=out_specs,
        out_shape=out_shape,
        compiler_params=pltpu.CompilerParams(
            dimension_semantics=("parallel",)),
    )(ge, pos3, wl1, wr, wl2, wd, r)


def _layer(tbl, halves, wl1, wr, wl2, wd, r, has_next):
    ges = [_sc_gather(tbl, idx_h).reshape(_K, _H, tbl.shape[1])
           for idx_h, _ in halves]
    outs = []
    for ge, (_, pos_h) in zip(ges, halves):
        outs.append(_conv(ge, pos_h, wl1, wr, wl2, wd, r, has_next))
    parts = [jnp.concatenate([a, b], axis=0) for a, b in zip(*outs)]
    h = jnp.stack(parts[:3], axis=-1)
    return (h, parts[3]) if has_next else (h, None)


def kernel(pos, c1_l1, c1_r, c1_l2, c1_d, r1, c2_l1, c2_r, c2_l2, c2_d, r2,
           c3_l1, c3_r, c3_l2, c3_d, r3):
    pos3 = pos[:, 0, :]
    posT = pos3.T

    t1 = _mk_t1(pos3)
    halves = tuple(
        (_knn(pos3[h * _H:(h + 1) * _H], posT).T.reshape(-1),
         pos3[h * _H:(h + 1) * _H])
        for h in range(2))

    h1, t2 = _layer(t1, halves, c1_l1.T, c1_r.T, c1_l2.T, c1_d.T, r1.T, True)
    h2, t3 = _layer(t2, halves, c2_l1.T, c2_r.T, c2_l2.T, c2_d.T, r2.T, True)
    h3, _ = _layer(t3, halves, c3_l1.T, c3_r.T, c3_l2.T, c3_d.T, r3.T, False)
    return (h1, h2, h3)
